# baseline (speedup 1.0000x reference)
import jax
import jax.numpy as jnp
from jax.experimental import pallas as pl

AXON = 1000
NEURON = 10000
T_STEPS = 16
BETA = 0.95
THRESH = 1.0
B = 128

NT = 512


def _lif_body(x_ref, w_ref, m8_ref, b_ref, spk_ref, mem_ref):
    m8 = m8_ref[...].astype(jnp.int32)
    e = jnp.broadcast_to(m8[:, None, :], (NT // 8, 8, AXON))
    e = e.reshape(NT, AXON)
    sh = 7 - (jax.lax.broadcasted_iota(jnp.int32, (NT, AXON), 0) % 8)
    bits = jnp.right_shift(e, sh) & 1
    wm = w_ref[...] * bits.astype(jnp.float32)
    cur = jax.lax.dot_general(
        x_ref[...], wm,
        dimension_numbers=(((1,), (1,)), ((), ())),
        preferred_element_type=jnp.float32,
    ) + b_ref[...]
    mem = cur
    spk = (mem > THRESH).astype(jnp.float32)
    spk_ref[0] = spk
    mem_ref[0] = mem
    for t in range(1, T_STEPS):
        mem = BETA * mem + cur - spk * THRESH
        spk = (mem > THRESH).astype(jnp.float32)
        spk_ref[t] = spk
        mem_ref[t] = mem


def kernel(x, W, b, mask):
    b2 = b.reshape(1, NEURON)
    m8 = jnp.packbits(mask, axis=0)
    grid = (NEURON // NT + (NEURON % NT > 0),)
    spk, mem = pl.pallas_call(
        _lif_body,
        grid=grid,
        in_specs=[
            pl.BlockSpec((B, AXON), lambda i: (0, 0)),
            pl.BlockSpec((NT, AXON), lambda i: (i, 0)),
            pl.BlockSpec((NT // 8, AXON), lambda i: (i, 0)),
            pl.BlockSpec((1, NT), lambda i: (0, i)),
        ],
        out_specs=[
            pl.BlockSpec((T_STEPS, B, NT), lambda i: (0, 0, i)),
            pl.BlockSpec((T_STEPS, B, NT), lambda i: (0, 0, i)),
        ],
        out_shape=[
            jax.ShapeDtypeStruct((T_STEPS, B, NEURON), jnp.float32),
            jax.ShapeDtypeStruct((T_STEPS, B, NEURON), jnp.float32),
        ],
    )(x, W, m8, b2)
    return spk, mem


# speedup vs baseline: 1.0069x; 1.0069x over previous
import jax
import jax.numpy as jnp
from jax.experimental import pallas as pl

AXON = 1000
NEURON = 10000
T_STEPS = 16
BETA = 0.95
THRESH = 1.0
B = 128

NT = 1024


def _lif_body(x_ref, w_ref, m8_ref, b_ref, spk_ref, mem_ref):
    m8 = m8_ref[...].astype(jnp.int32)
    e = jnp.broadcast_to(m8[:, None, :], (NT // 8, 8, AXON))
    e = e.reshape(NT, AXON)
    sh = 7 - (jax.lax.broadcasted_iota(jnp.int32, (NT, AXON), 0) % 8)
    bits = jnp.right_shift(e, sh) & 1
    wm = w_ref[...] * bits.astype(jnp.float32)
    cur = jax.lax.dot_general(
        x_ref[...], wm,
        dimension_numbers=(((1,), (1,)), ((), ())),
        preferred_element_type=jnp.float32,
    ) + b_ref[...]
    mem = cur
    spk = (mem > THRESH).astype(jnp.float32)
    spk_ref[0] = spk
    mem_ref[0] = mem
    for t in range(1, T_STEPS):
        mem = BETA * mem + cur - spk * THRESH
        spk = (mem > THRESH).astype(jnp.float32)
        spk_ref[t] = spk
        mem_ref[t] = mem


def kernel(x, W, b, mask):
    b2 = b.reshape(1, NEURON)
    m8 = jnp.packbits(mask, axis=0)
    grid = (NEURON // NT + (NEURON % NT > 0),)
    spk, mem = pl.pallas_call(
        _lif_body,
        grid=grid,
        in_specs=[
            pl.BlockSpec((B, AXON), lambda i: (0, 0)),
            pl.BlockSpec((NT, AXON), lambda i: (i, 0)),
            pl.BlockSpec((NT // 8, AXON), lambda i: (i, 0)),
            pl.BlockSpec((1, NT), lambda i: (0, i)),
        ],
        out_specs=[
            pl.BlockSpec((T_STEPS, B, NT), lambda i: (0, 0, i)),
            pl.BlockSpec((T_STEPS, B, NT), lambda i: (0, 0, i)),
        ],
        out_shape=[
            jax.ShapeDtypeStruct((T_STEPS, B, NEURON), jnp.float32),
            jax.ShapeDtypeStruct((T_STEPS, B, NEURON), jnp.float32),
        ],
    )(x, W, m8, b2)
    return spk, mem


# speedup vs baseline: 1.0393x; 1.0322x over previous
import jax
import jax.numpy as jnp
from jax.experimental import pallas as pl

AXON = 1000
NEURON = 10000
T_STEPS = 16
BETA = 0.95
THRESH = 1.0
B = 128

NT = 1024


def _lif_body(x_ref, w_ref, m8_ref, b_ref, spk_ref, mem_ref):
    m8 = m8_ref[...].astype(jnp.int32)
    e = jnp.broadcast_to(m8[:, None, :], (NT // 8, 8, AXON))
    e = e.reshape(NT, AXON)
    sh = 7 - (jax.lax.broadcasted_iota(jnp.int32, (NT, AXON), 0) % 8)
    bits = jnp.right_shift(e, sh) & 1
    wm = w_ref[...] * bits.astype(jnp.float32)
    cur = jax.lax.dot_general(
        x_ref[...], wm,
        dimension_numbers=(((1,), (1,)), ((), ())),
        preferred_element_type=jnp.float32,
    ) + b_ref[...]
    mem = cur
    spk = (mem > THRESH).astype(jnp.float32)
    spk_ref[0] = spk
    mem_ref[0] = mem
    for t in range(1, T_STEPS):
        mem = BETA * mem + cur - spk * THRESH
        spk = (mem > THRESH).astype(jnp.float32)
        spk_ref[t] = spk
        mem_ref[t] = mem


def kernel(x, W, b, mask):
    b2 = b.reshape(1, NEURON)
    shifts = (7 - jnp.arange(8, dtype=jnp.int32))[None, :, None]
    m8 = (mask.reshape(NEURON // 8, 8, AXON).astype(jnp.int32) << shifts
          ).sum(axis=1).astype(jnp.uint8)
    grid = (NEURON // NT + (NEURON % NT > 0),)
    spk, mem = pl.pallas_call(
        _lif_body,
        grid=grid,
        in_specs=[
            pl.BlockSpec((B, AXON), lambda i: (0, 0)),
            pl.BlockSpec((NT, AXON), lambda i: (i, 0)),
            pl.BlockSpec((NT // 8, AXON), lambda i: (i, 0)),
            pl.BlockSpec((1, NT), lambda i: (0, i)),
        ],
        out_specs=[
            pl.BlockSpec((T_STEPS, B, NT), lambda i: (0, 0, i)),
            pl.BlockSpec((T_STEPS, B, NT), lambda i: (0, 0, i)),
        ],
        out_shape=[
            jax.ShapeDtypeStruct((T_STEPS, B, NEURON), jnp.float32),
            jax.ShapeDtypeStruct((T_STEPS, B, NEURON), jnp.float32),
        ],
    )(x, W, m8, b2)
    return spk, mem


# speedup vs baseline: 1.0409x; 1.0015x over previous
import jax
import jax.numpy as jnp
from jax.experimental import pallas as pl

AXON = 1000
NEURON = 10000
T_STEPS = 16
BETA = 0.95
THRESH = 1.0
B = 128

NT = 1280


def _lif_body(x_ref, w_ref, m8_ref, b_ref, spk_ref, mem_ref):
    m8 = m8_ref[...].astype(jnp.int32)
    e = jnp.broadcast_to(m8[:, None, :], (NT // 8, 8, AXON))
    e = e.reshape(NT, AXON)
    sh = 7 - (jax.lax.broadcasted_iota(jnp.int32, (NT, AXON), 0) % 8)
    bits = jnp.right_shift(e, sh) & 1
    wm = w_ref[...] * bits.astype(jnp.float32)
    cur = jax.lax.dot_general(
        x_ref[...], wm,
        dimension_numbers=(((1,), (1,)), ((), ())),
        preferred_element_type=jnp.float32,
    ) + b_ref[...]
    mem = cur
    spk = (mem > THRESH).astype(jnp.float32)
    spk_ref[0] = spk
    mem_ref[0] = mem
    for t in range(1, T_STEPS):
        mem = BETA * mem + cur - spk * THRESH
        spk = (mem > THRESH).astype(jnp.float32)
        spk_ref[t] = spk
        mem_ref[t] = mem


def kernel(x, W, b, mask):
    b2 = b.reshape(1, NEURON)
    shifts = (7 - jnp.arange(8, dtype=jnp.int32))[None, :, None]
    m8 = (mask.reshape(NEURON // 8, 8, AXON).astype(jnp.int32) << shifts
          ).sum(axis=1).astype(jnp.uint8)
    grid = (NEURON // NT + (NEURON % NT > 0),)
    spk, mem = pl.pallas_call(
        _lif_body,
        grid=grid,
        in_specs=[
            pl.BlockSpec((B, AXON), lambda i: (0, 0)),
            pl.BlockSpec((NT, AXON), lambda i: (i, 0)),
            pl.BlockSpec((NT // 8, AXON), lambda i: (i, 0)),
            pl.BlockSpec((1, NT), lambda i: (0, i)),
        ],
        out_specs=[
            pl.BlockSpec((T_STEPS, B, NT), lambda i: (0, 0, i)),
            pl.BlockSpec((T_STEPS, B, NT), lambda i: (0, 0, i)),
        ],
        out_shape=[
            jax.ShapeDtypeStruct((T_STEPS, B, NEURON), jnp.float32),
            jax.ShapeDtypeStruct((T_STEPS, B, NEURON), jnp.float32),
        ],
    )(x, W, m8, b2)
    return spk, mem
